# baseline (speedup 1.0000x reference)
import functools

import jax
import jax.numpy as jnp
from jax import lax
from jax.experimental import pallas as pl
from jax.experimental.pallas import tpu as pltpu
from jax.experimental.pallas import tpu_sc as plsc

C = 19
P = 1048576
NB = 1024
NC, NS, L = 2, 16, 16
NW = NC * NS
PIX_PER_TILE = P // NW
CHUNK_PIX = 1024
N_CHUNKS = PIX_PER_TILE // CHUNK_PIX
CHUNK_ELEMS = CHUNK_PIX * C
HROWS = C + 1
HCOLS = 2 * NB


def _sc_body(x_hbm, t_hbm, out_hbm, xb0, tb0, xb1, tb1,
             hist, sx0, st0, sx1, st1):
    cid = lax.axis_index("c")
    sid = lax.axis_index("s")
    wid = sid * NC + cid
    iota = lax.broadcasted_iota(jnp.int32, (L,), 0)
    zeros = jnp.zeros((L,), jnp.int32)

    def zero_body(i, carry):
        r = i >> 4
        col = (i & 15) * (L * 8)
        for j in range(8):
            hist[r, pl.ds(col + j * L, L)] = zeros
        return carry

    lax.fori_loop(0, HROWS * HCOLS // (L * 8), zero_body, 0)

    pixbase = wid * PIX_PER_TILE

    def start(ch, xb, tb, sx, st):
        p0 = pixbase + ch * CHUNK_PIX
        dx = pltpu.async_copy(x_hbm.at[:, pl.ds(p0, CHUNK_PIX)], xb, sx)
        dt = pltpu.async_copy(t_hbm.at[pl.ds(p0, CHUNK_PIX)], tb, st)
        return dx, dt

    def drain(xb, tb, sx, st):
        pltpu.make_async_copy(x_hbm.at[:, pl.ds(0, CHUNK_PIX)], xb, sx).wait()
        pltpu.make_async_copy(t_hbm.at[pl.ds(0, CHUNK_PIX)], tb, st).wait()

    def compute(xb, tb):
        @plsc.parallel_loop(0, CHUNK_PIX // L)
        def pab(i):
            sl = pl.ds(i * L, L)
            evs = [jnp.exp(xb[c, sl]) for c in range(C)]
            tier = list(evs)
            while len(tier) > 1:
                nxt = [a + b for a, b in zip(tier[::2], tier[1::2])]
                if len(tier) % 2:
                    nxt[-1] = nxt[-1] + tier[-1]
                tier = nxt
            inv = 1.0 / tier[0]
            tg = tb[sl]
            for c in range(C):
                prob = evs[c] * inv
                isfg = tg == c
                err = jnp.where(isfg, 1.0 - prob, prob)
                b = jnp.minimum((err * float(NB)).astype(jnp.int32), NB - 1)
                col = jnp.where(isfg, NB, 0) + b
                cnt, last = plsc.scan_count(col)
                plsc.addupdate_scatter(hist, [iota * 0 + c, col], cnt,
                                       mask=last)

    start(0, xb0, tb0, sx0, st0)

    def chunk_pair(g, carry):
        d1x, d1t = start(2 * g + 1, xb1, tb1, sx1, st1)
        drain(xb0, tb0, sx0, st0)
        compute(xb0, tb0)

        @pl.when(g + 1 < N_CHUNKS // 2)
        def _prefetch():
            start(2 * g + 2, xb0, tb0, sx0, st0)

        d1x.wait()
        d1t.wait()
        compute(xb1, tb1)
        return carry

    lax.fori_loop(0, N_CHUNKS // 2, chunk_pair, 0)
    pltpu.sync_copy(hist, out_hbm.at[wid])


_sc_hist = functools.partial(
    pl.kernel,
    out_type=jax.ShapeDtypeStruct((NW, HROWS, HCOLS), jnp.int32),
    mesh=plsc.VectorSubcoreMesh(core_axis_name="c", subcore_axis_name="s"),
    compiler_params=pltpu.CompilerParams(needs_layout_passes=False),
    scratch_types=[
        pltpu.VMEM((C, CHUNK_PIX), jnp.float32),
        pltpu.VMEM((CHUNK_PIX,), jnp.int32),
        pltpu.VMEM((C, CHUNK_PIX), jnp.float32),
        pltpu.VMEM((CHUNK_PIX,), jnp.int32),
        pltpu.VMEM((HROWS, HCOLS), jnp.int32),
        pltpu.SemaphoreType.DMA,
        pltpu.SemaphoreType.DMA,
        pltpu.SemaphoreType.DMA,
        pltpu.SemaphoreType.DMA,
    ],
)(_sc_body)


def _suffix_excl(a):
    s = a
    k = 1
    while k < NB:
        rows = s.shape[0]
        s = s + jnp.concatenate(
            [s[:, k:], jnp.zeros((rows, k), jnp.float32)], axis=1)
        k *= 2
    return s - a


def _tc_body(h_ref, o_ref):
    a = h_ref[...].astype(jnp.float32)
    acc = jnp.sum(a, axis=0)[:C]
    m = acc[:, NB:]
    n = acc[:, :NB] + m
    if True:
        both = _suffix_excl(jnp.concatenate([n, m], axis=0))
        k0 = both[:C]
        s0 = both[C:]
        g = jnp.sum(m, axis=1, keepdims=True)
        d = jnp.maximum(g + k0 - s0, 1.0)
        nbg = n - m
        q = (lax.broadcasted_iota(jnp.int32, (C, NB), 1).astype(jnp.float32)
             + 0.5) / float(NB)
        fgc = q * m / d
        bgc = q * (g - s0 - m) * nbg / (d * (d + nbg))
        loss_c = jnp.sum(fgc + bgc, axis=1, keepdims=True)
        qmax = jnp.max(jnp.where(n > 0.0, q, -1.0), axis=1, keepdims=True)
        loss_c = jnp.where(g > 0.0, loss_c, qmax)
        o_ref[...] = jnp.sum(loss_c, axis=0, keepdims=True) / float(C)


_tc_reduce = pl.pallas_call(
    _tc_body,
    out_shape=jax.ShapeDtypeStruct((1, 1), jnp.float32),
)


def kernel(inputs, targets):
    t = targets.astype(jnp.int32)
    hists = _sc_hist(inputs.T, t)
    return _tc_reduce(hists)[0, 0]


# speedup vs baseline: 1.1861x; 1.1861x over previous
import functools

import jax
import jax.numpy as jnp
from jax import lax
from jax.experimental import pallas as pl
from jax.experimental.pallas import tpu as pltpu
from jax.experimental.pallas import tpu_sc as plsc

C = 19
P = 1048576
NB = 2048
NC, NS, L = 2, 16, 16
NW = NC * NS
PIX_PER_TILE = P // NW
CHUNK_PIX = 1024
N_CHUNKS = PIX_PER_TILE // CHUNK_PIX
CHUNK_ELEMS = CHUNK_PIX * C
HSIZE = 2 * C * NB


def _sc_body(x_hbm, t_hbm, out_hbm, xb0, tb0, xb1, tb1,
             hist, sx0, st0, sx1, st1):
    cid = lax.axis_index("c")
    sid = lax.axis_index("s")
    wid = sid * NC + cid
    zeros = jnp.zeros((L,), jnp.int32)

    def zero_body(i, carry):
        for j in range(8):
            hist[pl.ds(i * (L * 8) + j * L, L)] = zeros
        return carry

    lax.fori_loop(0, HSIZE // (L * 8), zero_body, 0)

    pixbase = wid * PIX_PER_TILE

    def start(ch, xb, tb, sx, st):
        p0 = pixbase + ch * CHUNK_PIX
        dx = pltpu.async_copy(x_hbm.at[:, pl.ds(p0, CHUNK_PIX)], xb, sx)
        dt = pltpu.async_copy(t_hbm.at[pl.ds(p0, CHUNK_PIX)], tb, st)
        return dx, dt

    def drain(xb, tb, sx, st):
        pltpu.make_async_copy(x_hbm.at[:, pl.ds(0, CHUNK_PIX)], xb, sx).wait()
        pltpu.make_async_copy(t_hbm.at[pl.ds(0, CHUNK_PIX)], tb, st).wait()

    def compute(xb, tb):
        @plsc.parallel_loop(0, CHUNK_PIX // L)
        def pab(i):
            sl = pl.ds(i * L, L)
            evs = [jnp.exp(xb[c, sl]) for c in range(C)]
            tier = list(evs)
            while len(tier) > 1:
                nxt = [a + b for a, b in zip(tier[::2], tier[1::2])]
                if len(tier) % 2:
                    nxt[-1] = nxt[-1] + tier[-1]
                tier = nxt
            inv = 1.0 / tier[0]
            tg = tb[sl]
            for c in range(C):
                prob = evs[c] * inv
                isfg = tg == c
                err = jnp.where(isfg, 1.0 - prob, prob)
                b = jnp.minimum((err * float(NB)).astype(jnp.int32), NB - 1)
                idx = jnp.where(isfg, (C + c) * NB, c * NB) + b
                cnt, last = plsc.scan_count(idx)
                plsc.addupdate_scatter(hist, [idx], cnt, mask=last)

    start(0, xb0, tb0, sx0, st0)

    def chunk_pair(g, carry):
        d1x, d1t = start(2 * g + 1, xb1, tb1, sx1, st1)
        drain(xb0, tb0, sx0, st0)
        compute(xb0, tb0)

        @pl.when(g + 1 < N_CHUNKS // 2)
        def _prefetch():
            start(2 * g + 2, xb0, tb0, sx0, st0)

        d1x.wait()
        d1t.wait()
        compute(xb1, tb1)
        return carry

    lax.fori_loop(0, N_CHUNKS // 2, chunk_pair, 0)
    pltpu.sync_copy(hist, out_hbm.at[wid])


_sc_hist = functools.partial(
    pl.kernel,
    out_type=jax.ShapeDtypeStruct((NW, HSIZE), jnp.int32),
    mesh=plsc.VectorSubcoreMesh(core_axis_name="c", subcore_axis_name="s"),
    compiler_params=pltpu.CompilerParams(needs_layout_passes=False),
    scratch_types=[
        pltpu.VMEM((C, CHUNK_PIX), jnp.float32),
        pltpu.VMEM((CHUNK_PIX,), jnp.int32),
        pltpu.VMEM((C, CHUNK_PIX), jnp.float32),
        pltpu.VMEM((CHUNK_PIX,), jnp.int32),
        pltpu.VMEM((HSIZE,), jnp.int32),
        pltpu.SemaphoreType.DMA,
        pltpu.SemaphoreType.DMA,
        pltpu.SemaphoreType.DMA,
        pltpu.SemaphoreType.DMA,
    ],
)(_sc_body)


def _suffix_excl(a):
    s = a
    k = 1
    while k < NB:
        rows = s.shape[0]
        s = s + jnp.concatenate(
            [s[:, k:], jnp.zeros((rows, k), jnp.float32)], axis=1)
        k *= 2
    return s - a


def _tc_body(h_ref, o_ref, acc_ref):
    i = pl.program_id(0)

    @pl.when(i == 0)
    def _zero():
        acc_ref[...] = jnp.zeros_like(acc_ref)

    acc_ref[...] += h_ref[...][0].astype(jnp.float32)

    @pl.when(i == NW - 1)
    def _finish():
        a = acc_ref[...]
        m = a[1]
        n = a[0] + m
        both = _suffix_excl(jnp.concatenate([n, m], axis=0))
        k0 = both[:C]
        s0 = both[C:]
        g = jnp.sum(m, axis=1, keepdims=True)
        d = jnp.maximum(g + k0 - s0, 1.0)
        nbg = n - m
        q = (lax.broadcasted_iota(jnp.int32, (C, NB), 1).astype(jnp.float32)
             + 0.5) / float(NB)
        fgc = q * m / d
        bgc = q * (g - s0 - m) * nbg / (d * (d + nbg))
        loss_c = jnp.sum(fgc + bgc, axis=1, keepdims=True)
        qmax = jnp.max(jnp.where(n > 0.0, q, -1.0), axis=1, keepdims=True)
        loss_c = jnp.where(g > 0.0, loss_c, qmax)
        o_ref[...] = jnp.sum(loss_c, axis=0, keepdims=True) / float(C)


_tc_reduce = pl.pallas_call(
    _tc_body,
    grid=(NW,),
    in_specs=[pl.BlockSpec((1, 2, C, NB), lambda i: (i, 0, 0, 0))],
    out_specs=pl.BlockSpec((1, 1), lambda i: (0, 0)),
    out_shape=jax.ShapeDtypeStruct((1, 1), jnp.float32),
    scratch_shapes=[pltpu.VMEM((2, C, NB), jnp.float32)],
)


def kernel(inputs, targets):
    t = targets.astype(jnp.int32)
    hists = _sc_hist(inputs.T, t)
    h = hists.reshape(NW, 2, C, NB)
    return _tc_reduce(h)[0, 0]
